# Initial kernel scaffold; baseline (speedup 1.0000x reference)
import functools

import jax
import jax.numpy as jnp
from jax import lax
from jax.experimental import pallas as pl
from jax.experimental.pallas import tpu as pltpu
from jax.experimental.pallas import tpu_sc as plsc

NC = 2
NS = 16
NWORK = NC * NS
CH = 128
NUM_GRAPHS = 128


def _deg_body(nchunk, ep_w, rpt, dst, zeros16, ones16, out, idx_d, ones_v, acc):
    c = lax.axis_index("c")
    s = lax.axis_index("s")
    wid = c * NS + s
    base_r = s * rpt
    pltpu.sync_copy(zeros16.at[pl.ds(base_r, rpt)], acc.at[pl.ds(base_r, rpt)])
    pltpu.sync_copy(ones16, ones_v)
    plsc.subcore_barrier()
    ebase = wid * ep_w

    def chunk(i, carry):
        off = pl.multiple_of(ebase + i * CH, CH)
        pltpu.sync_copy(dst.at[pl.ds(off, CH)], idx_d)
        pltpu.sync_copy(ones_v, acc.at[idx_d], add=True)
        return carry

    lax.fori_loop(0, nchunk, chunk, 0)
    plsc.subcore_barrier()
    pltpu.sync_copy(acc.at[pl.ds(base_r, rpt)], out.at[c, pl.ds(base_r, rpt)])


def _msg_body(nchunk, ep_w, rpt, hp, src, dst, zeros, out,
              idx_s, idx_d, rows, acc):
    c = lax.axis_index("c")
    s = lax.axis_index("s")
    wid = c * NS + s
    base_r = s * rpt
    pltpu.sync_copy(zeros.at[pl.ds(base_r, rpt)], acc.at[pl.ds(base_r, rpt)])
    plsc.subcore_barrier()
    ebase = wid * ep_w

    def chunk(i, carry):
        off = pl.multiple_of(ebase + i * CH, CH)
        pltpu.sync_copy(src.at[pl.ds(off, CH)], idx_s)
        pltpu.sync_copy(dst.at[pl.ds(off, CH)], idx_d)
        pltpu.sync_copy(hp.at[idx_s], rows)
        pltpu.sync_copy(rows, acc.at[idx_d], add=True)
        return carry

    lax.fori_loop(0, nchunk, chunk, 0)
    plsc.subcore_barrier()
    pltpu.sync_copy(acc.at[pl.ds(base_r, rpt)], out.at[c, pl.ds(base_r, rpt)])


def _tc0_body(n, degp_ref, x_ref, w1_ref, dinv_ref, hp1_ref):
    deg = 1.0 + degp_ref[0, 0:n, 0:1] + degp_ref[1, 0:n, 0:1]
    dinv = lax.rsqrt(deg)
    dinv_ref[...] = dinv
    h = jnp.dot(x_ref[...], w1_ref[...], preferred_element_type=jnp.float32)
    hp1_ref[...] = dinv * h


def _tcmid_body(n, p_ref, hp_ref, dinv_ref, b_ref, w_ref, out_ref):
    dinv = dinv_ref[...]
    tot = p_ref[0, 0:n, :] + p_ref[1, 0:n, :] + hp_ref[...]
    a = jnp.maximum(dinv * tot + b_ref[...], 0.0)
    out_ref[...] = dinv * jnp.dot(a, w_ref[...],
                                  preferred_element_type=jnp.float32)


def _tcfin_body(n, p_ref, hp_ref, dinv_ref, b3_ref, batch_ref, wl_ref,
                bl_ref, out_ref):
    dinv = dinv_ref[...]
    h3 = dinv * (p_ref[0, 0:n, :] + p_ref[1, 0:n, :] + hp_ref[...]) + b3_ref[...]
    gid = lax.broadcasted_iota(jnp.int32, (1, NUM_GRAPHS), 1)
    onehot = (batch_ref[...] == gid).astype(jnp.float32)
    sums = lax.dot_general(onehot, h3, (((0,), (0,)), ((), ())),
                           preferred_element_type=jnp.float32)
    cnts = jnp.sum(onehot, axis=0)[:, None]
    pooled = sums / jnp.maximum(cnts, 1.0)
    out_ref[...] = jnp.dot(pooled, wl_ref[...],
                           preferred_element_type=jnp.float32) + bl_ref[...]


def kernel(x, edge_index, batch, W1, b1, W2, b2, W3, b3, Wl, bl):
    n, d = x.shape
    h = W1.shape[1]
    c_out = Wl.shape[1]
    e = edge_index.shape[1]

    unit = NWORK * CH
    ep = ((e + unit - 1) // unit) * unit
    pad = ep - e
    ep_w = ep // NWORK
    nchunk = ep_w // CH
    npad = ((n + 1 + NS * 8 - 1) // (NS * 8)) * (NS * 8)
    rpt = npad // NS

    src = jnp.concatenate(
        [edge_index[0].astype(jnp.int32), jnp.zeros((pad,), jnp.int32)])
    dst = jnp.concatenate(
        [edge_index[1].astype(jnp.int32), jnp.full((pad,), n, jnp.int32)])
    zeros = jnp.zeros((npad, h), jnp.float32)
    zeros16 = jnp.zeros((npad, 16), jnp.float32)
    ones16 = jnp.ones((CH, 16), jnp.float32)
    batch2 = batch.astype(jnp.int32).reshape(n, 1)
    b1r, b2r, b3r = (b.reshape(1, -1) for b in (b1, b2, b3))
    blr = bl.reshape(1, -1)

    mesh = plsc.VectorSubcoreMesh(core_axis_name="c", subcore_axis_name="s",
                                  num_cores=NC, num_subcores=NS)

    degp = pl.kernel(
        functools.partial(_deg_body, nchunk, ep_w, rpt),
        out_type=jax.ShapeDtypeStruct((NC, npad, 16), jnp.float32),
        mesh=mesh,
        scratch_types=[
            pltpu.VMEM((CH,), jnp.int32),
            pltpu.VMEM((CH, 16), jnp.float32),
            pltpu.VMEM_SHARED((npad, 16), jnp.float32),
        ],
        name="sc_degree",
    )(dst, zeros16, ones16)

    def msg(hp):
        return pl.kernel(
            functools.partial(_msg_body, nchunk, ep_w, rpt),
            out_type=jax.ShapeDtypeStruct((NC, npad, h), jnp.float32),
            mesh=mesh,
            scratch_types=[
                pltpu.VMEM((CH,), jnp.int32),
                pltpu.VMEM((CH,), jnp.int32),
                pltpu.VMEM((CH, h), jnp.float32),
                pltpu.VMEM_SHARED((npad, h), jnp.float32),
            ],
            name="sc_message",
        )(hp, src, dst, zeros)

    f32 = jnp.float32
    dinv, hp1 = pl.pallas_call(
        functools.partial(_tc0_body, n),
        out_shape=[jax.ShapeDtypeStruct((n, 1), f32),
                   jax.ShapeDtypeStruct((n, h), f32)],
    )(degp, x, W1)

    p1 = msg(hp1)
    hp2 = pl.pallas_call(
        functools.partial(_tcmid_body, n),
        out_shape=jax.ShapeDtypeStruct((n, h), f32),
    )(p1, hp1, dinv, b1r, W2)

    p2 = msg(hp2)
    hp3 = pl.pallas_call(
        functools.partial(_tcmid_body, n),
        out_shape=jax.ShapeDtypeStruct((n, h), f32),
    )(p2, hp2, dinv, b2r, W3)

    p3 = msg(hp3)
    out = pl.pallas_call(
        functools.partial(_tcfin_body, n),
        out_shape=jax.ShapeDtypeStruct((NUM_GRAPHS, c_out), f32),
    )(p3, hp3, dinv, b3r, batch2, Wl, blr)
    return out


# speedup vs baseline: 9.5295x; 9.5295x over previous
import functools

import jax
import jax.numpy as jnp
from jax import lax
from jax.experimental import pallas as pl
from jax.experimental.pallas import tpu as pltpu
from jax.experimental.pallas import tpu_sc as plsc

NC = 2
NS = 16
NWORK = NC * NS
CH = 128
NUM_GRAPHS = 128


def _deg_body(nchunk, ep_w, rpt, dst, zeros, ones, out, idx_d, ones_v, acc):
    c = lax.axis_index("c")
    s = lax.axis_index("s")
    wid = c * NS + s
    base_r = s * rpt
    pltpu.sync_copy(zeros.at[pl.ds(base_r, rpt)], acc.at[pl.ds(base_r, rpt)])
    pltpu.sync_copy(ones, ones_v)
    plsc.subcore_barrier()
    ebase = wid * ep_w

    def chunk(i, carry):
        off = pl.multiple_of(ebase + i * CH, CH)
        pltpu.sync_copy(dst.at[pl.ds(off, CH)], idx_d)
        pltpu.sync_copy(ones_v, acc.at[idx_d], add=True)
        return carry

    lax.fori_loop(0, nchunk, chunk, 0)
    plsc.subcore_barrier()
    pltpu.sync_copy(acc.at[pl.ds(base_r, rpt)], out.at[c, pl.ds(base_r, rpt)])


def _msg_body(nchunk, ep_w, rpt, hp, src, dst, zeros, out,
              idx_s, idx_d, rows, acc):
    c = lax.axis_index("c")
    s = lax.axis_index("s")
    wid = c * NS + s
    base_r = s * rpt
    pltpu.sync_copy(zeros.at[pl.ds(base_r, rpt)], acc.at[pl.ds(base_r, rpt)])
    plsc.subcore_barrier()
    ebase = wid * ep_w

    def chunk(i, carry):
        off = pl.multiple_of(ebase + i * CH, CH)
        pltpu.sync_copy(src.at[pl.ds(off, CH)], idx_s)
        pltpu.sync_copy(dst.at[pl.ds(off, CH)], idx_d)
        pltpu.sync_copy(hp.at[idx_s], rows)
        pltpu.sync_copy(rows, acc.at[idx_d], add=True)
        return carry

    lax.fori_loop(0, nchunk, chunk, 0)
    plsc.subcore_barrier()
    pltpu.sync_copy(acc.at[pl.ds(base_r, rpt)], out.at[c, pl.ds(base_r, rpt)])


def _tc0_body(n, degp_ref, x_ref, w1_ref, dinv_ref, hp1_ref):
    deg = 1.0 + degp_ref[0, 0:n, 0:1] + degp_ref[1, 0:n, 0:1]
    dinv = lax.rsqrt(deg)
    dinv_ref[...] = dinv
    h = jnp.dot(x_ref[...], w1_ref[...], preferred_element_type=jnp.float32)
    hp1_ref[...] = dinv * h


def _tcmid_body(n, p_ref, hp_ref, dinv_ref, b_ref, w_ref, out_ref):
    dinv = dinv_ref[...]
    tot = p_ref[0, 0:n, :] + p_ref[1, 0:n, :] + hp_ref[...]
    a = jnp.maximum(dinv * tot + b_ref[...], 0.0)
    out_ref[...] = dinv * jnp.dot(a, w_ref[...],
                                  preferred_element_type=jnp.float32)


def _tcfin_body(n, p_ref, hp_ref, dinv_ref, b3_ref, batch_ref, wl_ref,
                bl_ref, out_ref):
    dinv = dinv_ref[...]
    h3 = dinv * (p_ref[0, 0:n, :] + p_ref[1, 0:n, :] + hp_ref[...]) + b3_ref[...]
    gid = lax.broadcasted_iota(jnp.int32, (1, NUM_GRAPHS), 1)
    onehot = (batch_ref[...] == gid).astype(jnp.float32)
    sums = lax.dot_general(onehot, h3, (((0,), (0,)), ((), ())),
                           preferred_element_type=jnp.float32)
    cnts = jnp.sum(onehot, axis=0)[:, None]
    pooled = sums / jnp.maximum(cnts, 1.0)
    out_ref[...] = jnp.dot(pooled, wl_ref[...],
                           preferred_element_type=jnp.float32) + bl_ref[...]


def kernel(x, edge_index, batch, W1, b1, W2, b2, W3, b3, Wl, bl):
    n, d = x.shape
    h = W1.shape[1]
    c_out = Wl.shape[1]
    e = edge_index.shape[1]

    unit = NWORK * CH
    ep = ((e + unit - 1) // unit) * unit
    pad = ep - e
    ep_w = ep // NWORK
    nchunk = ep_w // CH
    npad = ((n + 1 + NS * 8 - 1) // (NS * 8)) * (NS * 8)
    rpt = npad // NS

    src = jnp.concatenate(
        [edge_index[0].astype(jnp.int32), jnp.zeros((pad,), jnp.int32)])
    dst = jnp.concatenate(
        [edge_index[1].astype(jnp.int32), jnp.full((pad,), n, jnp.int32)])
    zeros = jnp.zeros((npad, h), jnp.float32)
    ones = jnp.ones((CH, h), jnp.float32)
    batch2 = batch.astype(jnp.int32).reshape(n, 1)
    b1r, b2r, b3r = (b.reshape(1, -1) for b in (b1, b2, b3))
    blr = bl.reshape(1, -1)

    mesh = plsc.VectorSubcoreMesh(core_axis_name="c", subcore_axis_name="s",
                                  num_cores=NC, num_subcores=NS)

    degp = pl.kernel(
        functools.partial(_deg_body, nchunk, ep_w, rpt),
        out_type=jax.ShapeDtypeStruct((NC, npad, h), jnp.float32),
        mesh=mesh,
        scratch_types=[
            pltpu.VMEM((CH,), jnp.int32),
            pltpu.VMEM((CH, h), jnp.float32),
            pltpu.VMEM_SHARED((npad, h), jnp.float32),
        ],
        name="sc_degree",
    )(dst, zeros, ones)

    def msg(hp):
        return pl.kernel(
            functools.partial(_msg_body, nchunk, ep_w, rpt),
            out_type=jax.ShapeDtypeStruct((NC, npad, h), jnp.float32),
            mesh=mesh,
            scratch_types=[
                pltpu.VMEM((CH,), jnp.int32),
                pltpu.VMEM((CH,), jnp.int32),
                pltpu.VMEM((CH, h), jnp.float32),
                pltpu.VMEM_SHARED((npad, h), jnp.float32),
            ],
            name="sc_message",
        )(hp, src, dst, zeros)

    f32 = jnp.float32
    dinv, hp1 = pl.pallas_call(
        functools.partial(_tc0_body, n),
        out_shape=[jax.ShapeDtypeStruct((n, 1), f32),
                   jax.ShapeDtypeStruct((n, h), f32)],
    )(degp, x, W1)

    p1 = msg(hp1)
    hp2 = pl.pallas_call(
        functools.partial(_tcmid_body, n),
        out_shape=jax.ShapeDtypeStruct((n, h), f32),
    )(p1, hp1, dinv, b1r, W2)

    p2 = msg(hp2)
    hp3 = pl.pallas_call(
        functools.partial(_tcmid_body, n),
        out_shape=jax.ShapeDtypeStruct((n, h), f32),
    )(p2, hp2, dinv, b2r, W3)

    p3 = msg(hp3)
    out = pl.pallas_call(
        functools.partial(_tcfin_body, n),
        out_shape=jax.ShapeDtypeStruct((NUM_GRAPHS, c_out), f32),
    )(p3, hp3, dinv, b3r, batch2, Wl, blr)
    return out
